# baseline (speedup 1.0000x reference)
import jax
import jax.numpy as jnp
import numpy as np
from jax.experimental import pallas as pl

N = 20000
C = 81
SCORE_THRESH = 0.05
NMS_THRESH = 0.5
DET_PER_IMG = 100
K = 200
IMG_W, IMG_H = 1333, 800
CLIP = float(np.log(1000.0 / 16.0))
WX, WY, WW, WH = 10.0, 10.0, 5.0, 5.0


def _decode(rel_codes, boxes):
    widths = boxes[:, 2] - boxes[:, 0] + 1.0
    heights = boxes[:, 3] - boxes[:, 1] + 1.0
    ctr_x = boxes[:, 0] + 0.5 * widths
    ctr_y = boxes[:, 1] + 0.5 * heights
    dx = rel_codes[:, 0::4] / WX
    dy = rel_codes[:, 1::4] / WY
    dw = jnp.minimum(rel_codes[:, 2::4] / WW, CLIP)
    dh = jnp.minimum(rel_codes[:, 3::4] / WH, CLIP)
    pred_ctr_x = dx * widths[:, None] + ctr_x[:, None]
    pred_ctr_y = dy * heights[:, None] + ctr_y[:, None]
    pred_w = jnp.exp(dw) * widths[:, None]
    pred_h = jnp.exp(dh) * heights[:, None]
    x1 = pred_ctr_x - 0.5 * pred_w
    y1 = pred_ctr_y - 0.5 * pred_h
    x2 = pred_ctr_x + 0.5 * pred_w - 1.0
    y2 = pred_ctr_y + 0.5 * pred_h - 1.0
    return jnp.stack([x1, y1, x2, y2], axis=2)


def _box_iou(b1, b2):
    area1 = (b1[:, 2] - b1[:, 0] + 1.0) * (b1[:, 3] - b1[:, 1] + 1.0)
    area2 = (b2[:, 2] - b2[:, 0] + 1.0) * (b2[:, 3] - b2[:, 1] + 1.0)
    lt = jnp.maximum(b1[:, None, :2], b2[None, :, :2])
    rb = jnp.minimum(b1[:, None, 2:], b2[None, :, 2:])
    wh = jnp.maximum(rb - lt + 1.0, 0.0)
    inter = wh[..., 0] * wh[..., 1]
    return inter / (area1[:, None] + area2[None, :] - inter)


def _nms_keep(boxes, valid):
    iou = _box_iou(boxes, boxes)
    idxs = jnp.arange(K)

    def body(i, keep):
        row = jax.lax.dynamic_slice_in_dim(iou, i, 1, axis=0)[0]
        alive_i = jax.lax.dynamic_slice_in_dim(keep, i, 1, axis=0)[0]
        sup = (row > NMS_THRESH) & alive_i & (idxs > i)
        return keep & (~sup)

    return jax.lax.fori_loop(0, K, body, valid)


def kernel(class_logits, box_regression, proposals):
    prob = jax.nn.softmax(class_logits, axis=-1)
    boxes = _decode(box_regression, proposals)
    x1 = jnp.clip(boxes[..., 0], 0.0, IMG_W - 1.0)
    y1 = jnp.clip(boxes[..., 1], 0.0, IMG_H - 1.0)
    x2 = jnp.clip(boxes[..., 2], 0.0, IMG_W - 1.0)
    y2 = jnp.clip(boxes[..., 3], 0.0, IMG_H - 1.0)
    boxes = jnp.stack([x1, y1, x2, y2], axis=-1)

    def per_class(scores_j, boxes_j):
        masked = jnp.where(scores_j > SCORE_THRESH, scores_j, -1.0)
        top_scores, idx = jax.lax.top_k(masked, K)
        top_boxes = jnp.take(boxes_j, idx, axis=0)
        valid = top_scores > SCORE_THRESH
        keep = _nms_keep(top_boxes, valid)
        out_scores = jnp.where(keep, top_scores, -1.0)
        return top_boxes, out_scores

    scores_t = jnp.transpose(prob[:, 1:])
    boxes_t = jnp.transpose(boxes[:, 1:, :], (1, 0, 2))
    cls_boxes, cls_scores = jax.vmap(per_class)(scores_t, boxes_t)

    all_scores = cls_scores.reshape(-1)
    all_boxes = cls_boxes.reshape(-1, 4)
    labels = jnp.repeat(jnp.arange(1, C, dtype=jnp.int64), K)

    final_scores, fidx = jax.lax.top_k(all_scores, DET_PER_IMG)
    final_boxes = jnp.take(all_boxes, fidx, axis=0)
    final_labels = jnp.take(labels, fidx, axis=0)
    dets = jnp.concatenate([final_boxes, final_scores[:, None]], axis=1)
    return dets, final_labels


# speedup vs baseline: 2.6658x; 2.6658x over previous
import functools

import jax
import jax.numpy as jnp
import numpy as np
from jax import lax
from jax.experimental import pallas as pl
from jax.experimental.pallas import tpu as pltpu
from jax.experimental.pallas import tpu_sc as plsc

N = 20000
C = 81
NCLS = C - 1
SCORE_THRESH = 0.05
NMS_THRESH = 0.5
DET_PER_IMG = 100
K = 200
IMG_W, IMG_H = 1333, 800
CLIP = float(np.log(1000.0 / 16.0))

NEG = np.float32(-3e38)
NCHUNK = N // 16


_B = 512
_GRID = (N + _B - 1) // _B


def _prep_body(lg_ref, rx_ref, ry_ref, rw_ref, rh_ref, prop_ref,
               probs_ref, x1_ref, y1_ref, x2_ref, y2_ref):
    lg = lg_ref[...]
    m = jnp.max(lg, axis=0, keepdims=True)
    e = jnp.exp(lg - m)
    s = jnp.sum(e, axis=0, keepdims=True)
    probs_ref[...] = (e / s)[1:, :]

    px1 = prop_ref[0:1, :]
    py1 = prop_ref[1:2, :]
    px2 = prop_ref[2:3, :]
    py2 = prop_ref[3:4, :]
    w = px2 - px1 + 1.0
    h = py2 - py1 + 1.0
    cx = px1 + 0.5 * w
    cy = py1 + 0.5 * h
    dx = rx_ref[...] / 10.0
    dy = ry_ref[...] / 10.0
    dw = jnp.minimum(rw_ref[...] / 5.0, CLIP)
    dh = jnp.minimum(rh_ref[...] / 5.0, CLIP)
    pcx = dx * w + cx
    pcy = dy * h + cy
    pw = jnp.exp(dw) * w
    ph = jnp.exp(dh) * h
    x1_ref[...] = jnp.clip(pcx - 0.5 * pw, 0.0, IMG_W - 1.0)
    y1_ref[...] = jnp.clip(pcy - 0.5 * ph, 0.0, IMG_H - 1.0)
    x2_ref[...] = jnp.clip(pcx + 0.5 * pw - 1.0, 0.0, IMG_W - 1.0)
    y2_ref[...] = jnp.clip(pcy + 0.5 * ph - 1.0, 0.0, IMG_H - 1.0)


def _tc_prep(logits_t, rx, ry, rw, rh, prop_t):
    spec81 = pl.BlockSpec((C, _B), lambda i: (0, i))
    return pl.pallas_call(
        _prep_body,
        grid=(_GRID,),
        in_specs=[spec81, spec81, spec81, spec81, spec81,
                  pl.BlockSpec((4, _B), lambda i: (0, i))],
        out_specs=[pl.BlockSpec((C - 1, _B), lambda i: (0, i)),
                   spec81, spec81, spec81, spec81],
        out_shape=[jax.ShapeDtypeStruct((C - 1, N), jnp.float32)] +
                  [jax.ShapeDtypeStruct((C, N), jnp.float32)] * 4,
    )(logits_t, rx, ry, rw, rh, prop_t)


def _iota16():
    return lax.broadcasted_iota(jnp.int32, (16,), 0)


def _splat_f(x):
    return jnp.broadcast_to(jnp.float32(x), (16,))


def _splat_i(x):
    return jnp.broadcast_to(jnp.int32(x), (16,))


def _lane0():
    return _iota16() == 0


def _count(mask):
    return jnp.sum(mask.astype(jnp.int32))


def _cstore(ref, base, x, mask):
    off = plsc.cumsum(mask.astype(jnp.int32)) - 1
    plsc.store_scatter(ref, [base + off], x, mask=mask)


def _store1(ref, pos, val):
    plsc.store_scatter(ref, [jnp.broadcast_to(pos, (16,))],
                       jnp.broadcast_to(val, (16,)), mask=_lane0())


def _bisect_bits(val_ref, n, rank):
    iota = _iota16()
    nv = (n + 15) >> 4

    def count_ge(cand):
        def body(kk, acc):
            v = val_ref[pl.ds(kk * 16, 16)]
            bits = plsc.bitcast(v, jnp.int32)
            m = (bits >= cand) & ((kk * 16 + iota) < n)
            return acc + _count(m)
        return lax.fori_loop(0, nv, body, jnp.int32(0))

    def bit_body(b, t):
        cand = t | jnp.left_shift(jnp.int32(1), 30 - b)
        cnt = count_ge(cand)
        return jnp.where(cnt >= rank, cand, t)

    return lax.fori_loop(0, 31, bit_body, jnp.int32(0))


def _sc_classes_body(probs_hbm, x1_hbm, y1_hbm, x2_hbm, y2_hbm,
                     scores_hbm, boxes_hbm,
                     probs_v, surv_val, surv_idx, nonsurv_idx,
                     sel_val, sel_idx, cand_idx, cand_score,
                     bx1, by1, bx2, by2, area, alive, score_out,
                     box4, idx2, sem):
    iota = _iota16()
    wid = lax.axis_index("s") * 2 + lax.axis_index("c")
    start = jnp.where(wid < 16, 3 * wid, 2 * wid + 16)
    cnt_cls = jnp.where(wid < 16, 3, 2)

    def per_class(jj, _):
        j = start + jj
        a = j + 1
        pltpu.sync_copy(probs_hbm.at[pl.ds(j * N, N)], probs_v.at[pl.ds(0, N)])

        def comp_body(kk, ptr):
            v = probs_v[pl.ds(kk * 16, 16)]
            m = v > SCORE_THRESH
            _cstore(surv_val, ptr, v, m)
            _cstore(surv_idx, ptr, kk * 16 + iota, m)
            return ptr + _count(m)
        n05 = lax.fori_loop(0, NCHUNK, comp_body, jnp.int32(0))

        t_bits = lax.cond(n05 > K,
                          lambda: _bisect_bits(surv_val, n05, K),
                          lambda: jnp.int32(0))

        def init_body(kk, _):
            sel_val[pl.ds(kk * 16, 16)] = _splat_f(NEG)
            return 0
        lax.fori_loop(0, 15, init_body, 0)

        def initc_body(kk, _):
            cand_idx[pl.ds(kk * 16, 16)] = _splat_i(0)
            cand_score[pl.ds(kk * 16, 16)] = _splat_f(-1.0)
            return 0
        lax.fori_loop(0, 14, initc_body, 0)

        nv = (n05 + 15) >> 4

        def sel_body(kk, ptr):
            v = surv_val[pl.ds(kk * 16, 16)]
            bits = plsc.bitcast(v, jnp.int32)
            m = (bits >= t_bits) & ((kk * 16 + iota) < n05)
            w = jnp.minimum(ptr, 208)
            _cstore(sel_val, w, v, m)
            _cstore(sel_idx, w, surv_idx[pl.ds(kk * 16, 16)], m)
            return ptr + _count(m)
        sptr = lax.fori_loop(0, nv, sel_body, jnp.int32(0))
        sel_n = jnp.minimum(sptr, K)

        def rank_body(r, _):
            @pl.when(r < sel_n)
            def _():
                def maxb(kk, m):
                    return jnp.maximum(m, jnp.max(sel_val[pl.ds(kk * 16, 16)]))
                mx = lax.fori_loop(0, 15, maxb, jnp.float32(NEG))

                def posb(kk, pm):
                    v = sel_val[pl.ds(kk * 16, 16)]
                    pos = jnp.where(v == mx, kk * 16 + iota, 240)
                    return jnp.minimum(pm, jnp.min(pos))
                p = lax.fori_loop(0, 15, posb, jnp.int32(240))

                idx_p = sel_idx[pl.ds(p, 16)][0]
                _store1(cand_score, r, mx)
                _store1(cand_idx, r, idx_p)
                _store1(sel_val, p, jnp.float32(NEG))
            return 0
        lax.fori_loop(0, K, rank_body, 0)

        @pl.when(sel_n < K)
        def _():
            def nons_body(kk, ptr):
                v = probs_v[pl.ds(kk * 16, 16)]
                m = ~(v > SCORE_THRESH)
                w = jnp.minimum(ptr, 208)
                _cstore(nonsurv_idx, w, kk * 16 + iota, m)
                return ptr + _count(m)
            lax.fori_loop(0, NCHUNK, nons_body, jnp.int32(0))

            def fill_body(t, _):
                g = t * 16 + iota
                m_fill = (g >= sel_n) & (g < K)
                fpos = jnp.clip(g - sel_n, 0, 223)
                vals = plsc.load_gather(nonsurv_idx, [fpos], mask=m_fill)
                plsc.store_scatter(cand_idx, [g], vals, mask=m_fill)
                return 0
            lax.fori_loop(0, 13, fill_body, 0)

        def hidx_body(t, _):
            g = t * 16 + iota
            ci = cand_idx[pl.ds(t * 16, 16)]
            plsc.store_scatter(idx2, [g // 112, g % 112], a * N + ci)
            return 0
        lax.fori_loop(0, 14, hidx_body, 0)

        copies = []
        for part in range(2):
            dst = pl.ds(part * 112, 112)
            copies.append(pltpu.async_copy(x1_hbm.at[idx2.at[part]], bx1.at[dst], sem))
            copies.append(pltpu.async_copy(y1_hbm.at[idx2.at[part]], by1.at[dst], sem))
            copies.append(pltpu.async_copy(x2_hbm.at[idx2.at[part]], bx2.at[dst], sem))
            copies.append(pltpu.async_copy(y2_hbm.at[idx2.at[part]], by2.at[dst], sem))
        for cp in copies:
            cp.wait()

        def area_body(t, _):
            d = pl.ds(t * 16, 16)
            area[d] = (bx2[d] - bx1[d] + 1.0) * (by2[d] - by1[d] + 1.0)
            alive[d] = (cand_score[d] > SCORE_THRESH).astype(jnp.int32)
            return 0
        lax.fori_loop(0, 14, area_body, 0)

        def nms_body(i, _):
            @pl.when(alive[pl.ds(i, 16)][0] != 0)
            def _():
                x1i = bx1[pl.ds(i, 16)][0]
                y1i = by1[pl.ds(i, 16)][0]
                x2i = bx2[pl.ds(i, 16)][0]
                y2i = by2[pl.ds(i, 16)][0]
                ai = area[pl.ds(i, 16)][0]

                def row_body(t, _):
                    d = pl.ds(t * 16, 16)
                    xx1 = jnp.maximum(x1i, bx1[d])
                    yy1 = jnp.maximum(y1i, by1[d])
                    xx2 = jnp.minimum(x2i, bx2[d])
                    yy2 = jnp.minimum(y2i, by2[d])
                    ww = jnp.maximum(xx2 - xx1 + 1.0, 0.0)
                    hh = jnp.maximum(yy2 - yy1 + 1.0, 0.0)
                    inter = ww * hh
                    iou = inter / (ai + area[d] - inter)
                    sup = (iou > NMS_THRESH) & ((t * 16 + iota) > i)
                    alive[d] = jnp.where(sup, 0, alive[d])
                    return 0
                lax.fori_loop(0, 14, row_body, 0)
            return 0
        lax.fori_loop(0, K, nms_body, 0)

        def out_body(t, _):
            d = pl.ds(t * 16, 16)
            g = t * 16 + iota
            score_out[d] = jnp.where(alive[d] != 0, cand_score[d], -1.0)
            plsc.store_scatter(box4, [g, _splat_i(0)], bx1[d])
            plsc.store_scatter(box4, [g, _splat_i(1)], by1[d])
            plsc.store_scatter(box4, [g, _splat_i(2)], bx2[d])
            plsc.store_scatter(box4, [g, _splat_i(3)], by2[d])
            return 0
        lax.fori_loop(0, 14, out_body, 0)

        pltpu.sync_copy(score_out.at[pl.ds(0, K)], scores_hbm.at[pl.ds(j * K, K)])
        pltpu.sync_copy(box4.at[pl.ds(0, K)], boxes_hbm.at[pl.ds(j * K, K)])
        return 0

    lax.fori_loop(0, cnt_cls, per_class, 0)


def _sc_classes(probs_flat, x1f, y1f, x2f, y2f):
    mesh = plsc.VectorSubcoreMesh(core_axis_name="c", subcore_axis_name="s")
    f32 = jnp.float32
    i32 = jnp.int32
    return pl.kernel(
        _sc_classes_body,
        out_type=[jax.ShapeDtypeStruct((NCLS * K,), f32),
                  jax.ShapeDtypeStruct((NCLS * K, 4), f32)],
        mesh=mesh,
        compiler_params=pltpu.CompilerParams(needs_layout_passes=False, use_tc_tiling_on_sc=False),
        scratch_types=[
            pltpu.VMEM((20016,), f32),
            pltpu.VMEM((20016,), f32),
            pltpu.VMEM((20016,), i32),
            pltpu.VMEM((240,), i32),
            pltpu.VMEM((240,), f32),
            pltpu.VMEM((240,), i32),
            pltpu.VMEM((224,), i32),
            pltpu.VMEM((224,), f32),
            pltpu.VMEM((224,), f32),
            pltpu.VMEM((224,), f32),
            pltpu.VMEM((224,), f32),
            pltpu.VMEM((224,), f32),
            pltpu.VMEM((224,), f32),
            pltpu.VMEM((224,), i32),
            pltpu.VMEM((224,), f32),
            pltpu.VMEM((224, 4), f32),
            pltpu.VMEM((2, 112), i32),
            pltpu.SemaphoreType.DMA,
        ],
    )(probs_flat, x1f, y1f, x2f, y2f)


def _sc_final_body(scores_hbm, boxes_hbm, dets_hbm, labels_hbm,
                   sc_v, surv_val, surv_idx, nonsurv_idx,
                   sel_val, sel_idx, fidx, fscore,
                   box_rows, idx1, dets_v, lab_v, sem):
    iota = _iota16()
    wid = lax.axis_index("s") * 2 + lax.axis_index("c")
    total = NCLS * K
    nch = total // 16

    @pl.when(wid == 0)
    def _():
        pltpu.sync_copy(scores_hbm, sc_v.at[pl.ds(0, total)])

        def comp_body(kk, ptr):
            v = sc_v[pl.ds(kk * 16, 16)]
            m = v > 0.0
            _cstore(surv_val, ptr, v, m)
            _cstore(surv_idx, ptr, kk * 16 + iota, m)
            return ptr + _count(m)
        pcnt = lax.fori_loop(0, nch, comp_body, jnp.int32(0))

        t_bits = lax.cond(pcnt > DET_PER_IMG,
                          lambda: _bisect_bits(surv_val, pcnt, DET_PER_IMG),
                          lambda: jnp.int32(0))

        def init_body(kk, _):
            sel_val[pl.ds(kk * 16, 16)] = _splat_f(NEG)
            return 0
        lax.fori_loop(0, 8, init_body, 0)

        def initf_body(kk, _):
            fidx[pl.ds(kk * 16, 16)] = _splat_i(0)
            fscore[pl.ds(kk * 16, 16)] = _splat_f(-1.0)
            return 0
        lax.fori_loop(0, 7, initf_body, 0)

        nv = (pcnt + 15) >> 4

        def sel_body(kk, ptr):
            v = surv_val[pl.ds(kk * 16, 16)]
            bits = plsc.bitcast(v, jnp.int32)
            m = (bits >= t_bits) & ((kk * 16 + iota) < pcnt)
            w = jnp.minimum(ptr, 112)
            _cstore(sel_val, w, v, m)
            _cstore(sel_idx, w, surv_idx[pl.ds(kk * 16, 16)], m)
            return ptr + _count(m)
        sptr = lax.fori_loop(0, nv, sel_body, jnp.int32(0))
        sel_n = jnp.minimum(sptr, DET_PER_IMG)

        def rank_body(r, _):
            @pl.when(r < sel_n)
            def _():
                def maxb(kk, m):
                    return jnp.maximum(m, jnp.max(sel_val[pl.ds(kk * 16, 16)]))
                mx = lax.fori_loop(0, 8, maxb, jnp.float32(NEG))

                def posb(kk, pm):
                    v = sel_val[pl.ds(kk * 16, 16)]
                    pos = jnp.where(v == mx, kk * 16 + iota, 128)
                    return jnp.minimum(pm, jnp.min(pos))
                p = lax.fori_loop(0, 8, posb, jnp.int32(128))

                idx_p = sel_idx[pl.ds(p, 16)][0]
                _store1(fscore, r, mx)
                _store1(fidx, r, idx_p)
                _store1(sel_val, p, jnp.float32(NEG))
            return 0
        lax.fori_loop(0, DET_PER_IMG, rank_body, 0)

        @pl.when(sel_n < DET_PER_IMG)
        def _():
            def nons_body(kk, ptr):
                v = sc_v[pl.ds(kk * 16, 16)]
                m = ~(v > 0.0)
                w = jnp.minimum(ptr, 112)
                _cstore(nonsurv_idx, w, kk * 16 + iota, m)
                return ptr + _count(m)
            lax.fori_loop(0, nch, nons_body, jnp.int32(0))

            def fill_body(t, _):
                g = t * 16 + iota
                m_fill = (g >= sel_n) & (g < DET_PER_IMG)
                fpos = jnp.clip(g - sel_n, 0, 127)
                vals = plsc.load_gather(nonsurv_idx, [fpos], mask=m_fill)
                plsc.store_scatter(fidx, [g], vals, mask=m_fill)
                return 0
            lax.fori_loop(0, 7, fill_body, 0)

        def lab_body(t, _):
            d = pl.ds(t * 16, 16)
            g = t * 16 + iota
            fi = fidx[d]
            lab_v[d] = fi // K + 1
            plsc.store_scatter(idx1, [g // 112, g % 112], fi)
            return 0
        lax.fori_loop(0, 7, lab_body, 0)

        pltpu.async_copy(boxes_hbm.at[idx1.at[0]], box_rows, sem).wait()

        def det_body(t, _):
            g = t * 16 + iota
            for col in range(4):
                v = plsc.load_gather(box_rows, [g, _splat_i(col)])
                plsc.store_scatter(dets_v, [g, _splat_i(col)], v)
            plsc.store_scatter(dets_v, [g, _splat_i(4)], fscore[pl.ds(t * 16, 16)])
            return 0
        lax.fori_loop(0, 7, det_body, 0)

        pltpu.sync_copy(dets_v.at[pl.ds(0, DET_PER_IMG)], dets_hbm)
        pltpu.sync_copy(lab_v.at[pl.ds(0, DET_PER_IMG)], labels_hbm)


def _sc_final(scores_flat, boxes_flat):
    mesh = plsc.VectorSubcoreMesh(core_axis_name="c", subcore_axis_name="s")
    f32 = jnp.float32
    i32 = jnp.int32
    return pl.kernel(
        _sc_final_body,
        out_type=[jax.ShapeDtypeStruct((DET_PER_IMG, 5), f32),
                  jax.ShapeDtypeStruct((DET_PER_IMG,), i32)],
        mesh=mesh,
        compiler_params=pltpu.CompilerParams(needs_layout_passes=False, use_tc_tiling_on_sc=False),
        scratch_types=[
            pltpu.VMEM((16000,), f32),
            pltpu.VMEM((16016,), f32),
            pltpu.VMEM((16016,), i32),
            pltpu.VMEM((128,), i32),
            pltpu.VMEM((144,), f32),
            pltpu.VMEM((144,), i32),
            pltpu.VMEM((112,), i32),
            pltpu.VMEM((112,), f32),
            pltpu.VMEM((112, 4), f32),
            pltpu.VMEM((1, 112), i32),
            pltpu.VMEM((112, 5), f32),
            pltpu.VMEM((112,), i32),
            pltpu.SemaphoreType.DMA,
        ],
    )(scores_flat, boxes_flat)


def kernel(class_logits, box_regression, proposals):
    logits_t = class_logits.T
    reg = box_regression.reshape(N, C, 4)
    rx = reg[:, :, 0].T
    ry = reg[:, :, 1].T
    rw = reg[:, :, 2].T
    rh = reg[:, :, 3].T
    prop_t = proposals.T

    probs_t, x1_t, y1_t, x2_t, y2_t = _tc_prep(logits_t, rx, ry, rw, rh, prop_t)

    scores_flat, boxes_flat = _sc_classes(
        probs_t.reshape(-1), x1_t.reshape(-1), y1_t.reshape(-1),
        x2_t.reshape(-1), y2_t.reshape(-1))

    dets, labels = _sc_final(scores_flat, boxes_flat)
    return dets, labels
